# Initial kernel scaffold; baseline (speedup 1.0000x reference)
import jax
import jax.numpy as jnp
from jax.experimental import pallas as pl


def kernel(shortest_paths, edge_feat, max_shortest_path_len, weight_embedding):
    raise NotImplementedError("write your pallas kernel here")


# speedup vs baseline: 13.2931x; 13.2931x over previous
import functools

import jax
import jax.numpy as jnp
from jax import lax
from jax.experimental import pallas as pl
from jax.experimental.pallas import tpu as pltpu
from jax.experimental.pallas import tpu_sc as plsc

MAX_PATH_LEN = 5
EDGE_FEAT_DIM = 128
NUM_HEADS = 32
N_NODES = 256
N_EDGES = 4096
B = N_NODES * N_NODES
P = 4104

NUM_CORES = 2
NUM_SUBCORES = 16
NW = NUM_CORES * NUM_SUBCORES
PAIRS_PER_W = B // NW
CHUNK = 128
NCHUNK = PAIRS_PER_W // CHUNK


def _proj_body(ef_ref, w_ref, out_ref):
    w = w_ref[0]
    mm = lax.dot_general(ef_ref[...], w, (((1,), (1,)), ((), ())),
                         preferred_element_type=jnp.float32)
    out_ref[0:N_EDGES, :] = mm
    out_ref[N_EDGES:P, :] = jnp.zeros((P - N_EDGES, NUM_HEADS), jnp.float32)


def _build_table(edge_feat, w):
    return pl.pallas_call(
        _proj_body,
        grid=(MAX_PATH_LEN,),
        in_specs=[
            pl.BlockSpec((N_EDGES, EDGE_FEAT_DIM), lambda l: (0, 0)),
            pl.BlockSpec((1, NUM_HEADS, EDGE_FEAT_DIM), lambda l: (l, 0, 0)),
        ],
        out_specs=pl.BlockSpec((P, NUM_HEADS), lambda l: (l, 0)),
        out_shape=jax.ShapeDtypeStruct((MAX_PATH_LEN * P, NUM_HEADS),
                                       jnp.float32),
    )(edge_feat, w)


@functools.partial(
    pl.kernel,
    out_type=jax.ShapeDtypeStruct((B, NUM_HEADS), jnp.float32),
    mesh=plsc.VectorSubcoreMesh(core_axis_name="c", subcore_axis_name="s"),
    compiler_params=pltpu.CompilerParams(use_tc_tiling_on_sc=False),
    scratch_types=[
        pltpu.VMEM((NCHUNK, MAX_PATH_LEN, CHUNK), jnp.int32),
        pltpu.VMEM((MAX_PATH_LEN * CHUNK, NUM_HEADS), jnp.float32),
        pltpu.VMEM((CHUNK, NUM_HEADS), jnp.float32),
        pltpu.SemaphoreType.DMA,
    ],
)
def _gather_accum(table_hbm, idx_hbm, out_hbm, idx_v, rows_v, out_v, sem):
    wid = lax.axis_index("s") * NUM_CORES + lax.axis_index("c")
    pltpu.sync_copy(idx_hbm.at[wid], idx_v)

    def chunk_body(c, carry):
        copies = [
            pltpu.async_copy(table_hbm.at[idx_v.at[c, l]],
                             rows_v.at[pl.ds(l * CHUNK, CHUNK)], sem)
            for l in range(MAX_PATH_LEN)
        ]
        for cp in copies:
            cp.wait()

        def acc_body(j, carry2):
            for h in range(NUM_HEADS // 16):
                sl = pl.ds(h * 16, 16)
                acc = rows_v[j, sl]
                for l in range(1, MAX_PATH_LEN):
                    acc = acc + rows_v[l * CHUNK + j, sl]
                out_v[j, sl] = acc
            return carry2

        lax.fori_loop(0, CHUNK, acc_body, 0, unroll=4)
        base = wid * PAIRS_PER_W + c * CHUNK
        pltpu.sync_copy(out_v, out_hbm.at[pl.ds(base, CHUNK)])
        return carry

    lax.fori_loop(0, NCHUNK, chunk_body, 0)


def kernel(shortest_paths, edge_feat, max_shortest_path_len, weight_embedding):
    mask = (jnp.arange(MAX_PATH_LEN)
            < jnp.minimum(MAX_PATH_LEN, max_shortest_path_len))
    w = weight_embedding[:MAX_PATH_LEN * NUM_HEADS].reshape(
        MAX_PATH_LEN, NUM_HEADS, EDGE_FEAT_DIM)
    w = w * mask.astype(w.dtype)[:, None, None]

    table = _build_table(edge_feat, w)

    sp = shortest_paths.reshape(B, MAX_PATH_LEN).astype(jnp.int32)
    idx = sp + (jnp.arange(MAX_PATH_LEN, dtype=jnp.int32) * P)[None, :]
    idx = idx.reshape(NW, NCHUNK, CHUNK, MAX_PATH_LEN).transpose(0, 1, 3, 2)

    out = _gather_accum(table, idx)
    return out.reshape(N_NODES, N_NODES, NUM_HEADS)


# speedup vs baseline: 13.8352x; 1.0408x over previous
import functools

import jax
import jax.numpy as jnp
from jax import lax
from jax.experimental import pallas as pl
from jax.experimental.pallas import tpu as pltpu
from jax.experimental.pallas import tpu_sc as plsc

MAX_PATH_LEN = 5
EDGE_FEAT_DIM = 128
NUM_HEADS = 32
N_NODES = 256
N_EDGES = 4096
B = N_NODES * N_NODES
P = 4128
PR = P * NUM_HEADS // 128
ER = N_EDGES * NUM_HEADS // 128

NUM_CORES = 2
NUM_SUBCORES = 16
NW = NUM_CORES * NUM_SUBCORES
PAIRS_PER_W = B // NW
CHUNK = 128
NCHUNK = PAIRS_PER_W // CHUNK
OUT_ROWS = CHUNK * NUM_HEADS // 128


def _proj_body(ef4_ref, wblk_ref, out_ref):
    mm = lax.dot_general(ef4_ref[...], wblk_ref[0],
                         (((1,), (0,)), ((), ())),
                         preferred_element_type=jnp.float32)
    out_ref[0:ER, :] = mm
    out_ref[ER:PR, :] = jnp.zeros((PR - ER, 128), jnp.float32)


def _build_table(ef4, wblk):
    return pl.pallas_call(
        _proj_body,
        grid=(MAX_PATH_LEN,),
        in_specs=[
            pl.BlockSpec((ER, 4 * EDGE_FEAT_DIM), lambda l: (0, 0)),
            pl.BlockSpec((1, 4 * EDGE_FEAT_DIM, 128), lambda l: (l, 0, 0)),
        ],
        out_specs=pl.BlockSpec((PR, 128), lambda l: (l, 0)),
        out_shape=jax.ShapeDtypeStruct((MAX_PATH_LEN * PR, 128), jnp.float32),
    )(ef4, wblk)


@functools.partial(
    pl.kernel,
    out_type=jax.ShapeDtypeStruct((B * NUM_HEADS // 128, 128), jnp.float32),
    mesh=plsc.VectorSubcoreMesh(core_axis_name="c", subcore_axis_name="s"),
    compiler_params=pltpu.CompilerParams(use_tc_tiling_on_sc=False),
    scratch_types=[
        pltpu.VMEM((NCHUNK * MAX_PATH_LEN, CHUNK), jnp.int32),
        pltpu.VMEM((MAX_PATH_LEN * CHUNK, NUM_HEADS), jnp.float32),
        pltpu.VMEM((OUT_ROWS, 128), jnp.float32),
        pltpu.SemaphoreType.DMA,
    ],
)
def _gather_accum(table_hbm, idx_hbm, out_hbm, idx_v, rows_v, out_v, sem):
    wid = lax.axis_index("s") * NUM_CORES + lax.axis_index("c")
    nrow = NCHUNK * MAX_PATH_LEN
    pltpu.sync_copy(idx_hbm.at[pl.ds(wid * nrow, nrow)], idx_v)

    def chunk_body(c, carry):
        copies = [
            pltpu.async_copy(table_hbm.at[idx_v.at[c * MAX_PATH_LEN + l]],
                             rows_v.at[pl.ds(l * CHUNK, CHUNK)], sem)
            for l in range(MAX_PATH_LEN)
        ]
        for cp in copies:
            cp.wait()

        def acc_body(j, carry2):
            row = lax.shift_right_logical(j, 2)
            colbase = lax.shift_left(lax.bitwise_and(j, 3), 5)
            for h in range(NUM_HEADS // 16):
                sl = pl.ds(h * 16, 16)
                acc = rows_v[j, sl]
                for l in range(1, MAX_PATH_LEN):
                    acc = acc + rows_v[l * CHUNK + j, sl]
                out_v[row, pl.ds(colbase + h * 16, 16)] = acc
            return carry2

        lax.fori_loop(0, CHUNK, acc_body, 0, unroll=4)
        base = wid * (PAIRS_PER_W * NUM_HEADS // 128) + c * OUT_ROWS
        pltpu.sync_copy(out_v, out_hbm.at[pl.ds(base, OUT_ROWS)])
        return carry

    lax.fori_loop(0, NCHUNK, chunk_body, 0)


def kernel(shortest_paths, edge_feat, max_shortest_path_len, weight_embedding):
    mask = (jnp.arange(MAX_PATH_LEN)
            < jnp.minimum(MAX_PATH_LEN, max_shortest_path_len))
    w = weight_embedding[:MAX_PATH_LEN * NUM_HEADS].reshape(
        MAX_PATH_LEN, NUM_HEADS, EDGE_FEAT_DIM)
    w = w * mask.astype(w.dtype)[:, None, None]
    wblk = jax.vmap(
        lambda m: jnp.kron(jnp.eye(4, dtype=m.dtype), m.T))(w)
    ef4 = edge_feat.reshape(ER, 4 * EDGE_FEAT_DIM)

    table = _build_table(ef4, wblk).reshape(MAX_PATH_LEN * P, NUM_HEADS)

    sp = shortest_paths.reshape(B, MAX_PATH_LEN).astype(jnp.int32)
    idx = sp + (jnp.arange(MAX_PATH_LEN, dtype=jnp.int32) * P)[None, :]
    idx = (idx.reshape(NW, NCHUNK, CHUNK, MAX_PATH_LEN)
              .transpose(0, 1, 3, 2)
              .reshape(NW * NCHUNK * MAX_PATH_LEN, CHUNK))

    out = _gather_accum(table, idx)
    return out.reshape(N_NODES, N_NODES, NUM_HEADS)


# speedup vs baseline: 14.6926x; 1.0620x over previous
import functools

import jax
import jax.numpy as jnp
from jax import lax
from jax.experimental import pallas as pl
from jax.experimental.pallas import tpu as pltpu
from jax.experimental.pallas import tpu_sc as plsc

MAX_PATH_LEN = 5
EDGE_FEAT_DIM = 128
NUM_HEADS = 32
N_NODES = 256
N_EDGES = 4096
B = N_NODES * N_NODES
EPR = 8
WPE = NUM_HEADS // 2
PR = 520
ER = N_EDGES // EPR
P = PR * EPR

NUM_CORES = 2
NUM_SUBCORES = 16
NW = NUM_CORES * NUM_SUBCORES
PAIRS_PER_W = B // NW
CHUNK = 128
NCHUNK = PAIRS_PER_W // CHUNK
OUT_ROWS = CHUNK * NUM_HEADS // 128


def _proj_body(ef8_ref, wblk_ref, out_ref):
    mm = lax.dot_general(ef8_ref[...], wblk_ref[0],
                         (((1,), (0,)), ((), ())),
                         preferred_element_type=jnp.float32)
    lo = lax.bitcast_convert_type(mm[:, 0:128], jnp.int32)
    hi = lax.bitcast_convert_type(mm[:, 128:256], jnp.int32)
    half = jnp.int32(0x8000)
    lo_b = jnp.bitwise_and(jnp.right_shift(lo + half, 16), jnp.int32(0xFFFF))
    hi_b = jnp.bitwise_and(hi + half, jnp.int32(-65536))
    out_ref[0:ER, :] = jnp.bitwise_or(lo_b, hi_b)
    out_ref[ER:PR, :] = jnp.zeros((PR - ER, 128), jnp.int32)


def _build_table(ef8, wblk):
    return pl.pallas_call(
        _proj_body,
        grid=(MAX_PATH_LEN,),
        in_specs=[
            pl.BlockSpec((ER, EPR * EDGE_FEAT_DIM), lambda l: (0, 0)),
            pl.BlockSpec((1, EPR * EDGE_FEAT_DIM, 256), lambda l: (l, 0, 0)),
        ],
        out_specs=pl.BlockSpec((PR, 128), lambda l: (l, 0)),
        out_shape=jax.ShapeDtypeStruct((MAX_PATH_LEN * PR, 128), jnp.int32),
    )(ef8, wblk)


@functools.partial(
    pl.kernel,
    out_type=jax.ShapeDtypeStruct((B * NUM_HEADS // 128, 128), jnp.float32),
    mesh=plsc.VectorSubcoreMesh(core_axis_name="c", subcore_axis_name="s"),
    compiler_params=pltpu.CompilerParams(use_tc_tiling_on_sc=False,
                                         needs_layout_passes=False),
    scratch_types=[
        pltpu.VMEM((NCHUNK * MAX_PATH_LEN, CHUNK), jnp.int32),
        pltpu.VMEM((MAX_PATH_LEN * CHUNK, WPE), jnp.int32),
        pltpu.VMEM((MAX_PATH_LEN * CHUNK, WPE), jnp.int32),
        pltpu.VMEM((OUT_ROWS, 128), jnp.float32),
        pltpu.VMEM((OUT_ROWS, 128), jnp.float32),
        pltpu.SemaphoreType.DMA,
        pltpu.SemaphoreType.DMA,
    ],
)
def _gather_accum(table_hbm, idx_hbm, out_hbm, idx_v,
                  rows_a, rows_b, out_a, out_b, sem_a, sem_b):
    wid = lax.axis_index("s") * NUM_CORES + lax.axis_index("c")
    nrow = NCHUNK * MAX_PATH_LEN
    pltpu.sync_copy(idx_hbm.at[pl.ds(wid * nrow, nrow)], idx_v)
    out_base = wid * (PAIRS_PER_W * NUM_HEADS // 128)

    def issue(c, rows_v, sem):
        for l in range(MAX_PATH_LEN):
            pltpu.async_copy(table_hbm.at[idx_v.at[c * MAX_PATH_LEN + l]],
                             rows_v.at[pl.ds(l * CHUNK, CHUNK)], sem)

    def drain(rows_v, sem):
        pltpu.make_async_copy(
            table_hbm.at[pl.ds(0, MAX_PATH_LEN * CHUNK)], rows_v, sem).wait()

    def accum(c, rows_v, out_v):
        himask = jnp.int32(-65536)

        def acc_body(j, carry2):
            row = lax.shift_right_logical(j, 2)
            colbase = lax.shift_left(lax.bitwise_and(j, 3), 5)
            acc_a = jnp.zeros((16,), jnp.float32)
            acc_b = jnp.zeros((16,), jnp.float32)
            for l in range(MAX_PATH_LEN):
                wv = rows_v[l * CHUNK + j, :]
                acc_a = acc_a + plsc.bitcast(jnp.left_shift(wv, 16),
                                             jnp.float32)
                acc_b = acc_b + plsc.bitcast(jnp.bitwise_and(wv, himask),
                                             jnp.float32)
            out_v[row, pl.ds(colbase, 16)] = acc_a
            out_v[row, pl.ds(colbase + 16, 16)] = acc_b
            return carry2

        lax.fori_loop(0, CHUNK, acc_body, 0, unroll=4)
        pltpu.sync_copy(out_v, out_hbm.at[pl.ds(out_base + c * OUT_ROWS,
                                                OUT_ROWS)])

    issue(0, rows_a, sem_a)

    def body(t, carry):
        c0 = 2 * t
        issue(c0 + 1, rows_b, sem_b)
        drain(rows_a, sem_a)
        accum(c0, rows_a, out_a)

        @pl.when(t < NCHUNK // 2 - 1)
        def _():
            issue(c0 + 2, rows_a, sem_a)

        drain(rows_b, sem_b)
        accum(c0 + 1, rows_b, out_b)
        return carry

    lax.fori_loop(0, NCHUNK // 2, body, 0)


def kernel(shortest_paths, edge_feat, max_shortest_path_len, weight_embedding):
    mask = (jnp.arange(MAX_PATH_LEN)
            < jnp.minimum(MAX_PATH_LEN, max_shortest_path_len))
    w = weight_embedding[:MAX_PATH_LEN * NUM_HEADS].reshape(
        MAX_PATH_LEN, NUM_HEADS, EDGE_FEAT_DIM)
    w = w * mask.astype(w.dtype)[:, None, None]

    eye8 = jnp.eye(EPR, dtype=w.dtype)
    wblk = jax.vmap(
        lambda m: jnp.concatenate(
            [jnp.kron(eye8, m.T[:, :16]), jnp.kron(eye8, m.T[:, 16:])],
            axis=1))(w)
    ef8 = edge_feat.reshape(ER, EPR * EDGE_FEAT_DIM)

    table = _build_table(ef8, wblk).reshape(MAX_PATH_LEN * P, WPE)

    sp = shortest_paths.reshape(B, MAX_PATH_LEN).astype(jnp.int32)
    idx = sp + (jnp.arange(MAX_PATH_LEN, dtype=jnp.int32) * P)[None, :]
    idx = (idx.reshape(NW, NCHUNK, CHUNK, MAX_PATH_LEN)
              .transpose(0, 1, 3, 2)
              .reshape(NW * NCHUNK * MAX_PATH_LEN, CHUNK))

    out = _gather_accum(table, idx)
    return out.reshape(N_NODES, N_NODES, NUM_HEADS)


# speedup vs baseline: 16.2206x; 1.1040x over previous
import functools

import jax
import jax.numpy as jnp
from jax import lax
from jax.experimental import pallas as pl
from jax.experimental.pallas import tpu as pltpu
from jax.experimental.pallas import tpu_sc as plsc

MAX_PATH_LEN = 5
EDGE_FEAT_DIM = 128
NUM_HEADS = 32
N_NODES = 256
N_EDGES = 4096
B = N_NODES * N_NODES
P = 4128
PR = P * NUM_HEADS // 128
ER = N_EDGES * NUM_HEADS // 128

NUM_CORES = 2
NUM_SUBCORES = 16
NW = NUM_CORES * NUM_SUBCORES
PAIRS_PER_W = B // NW
CHUNK = 128
NCHUNK = PAIRS_PER_W // CHUNK
OUT_ROWS = CHUNK * NUM_HEADS // 128


def _proj_body(ef4_ref, wblk_ref, out_ref):
    mm = lax.dot_general(ef4_ref[...], wblk_ref[0],
                         (((1,), (0,)), ((), ())),
                         preferred_element_type=jnp.float32)
    out_ref[0:ER, :] = mm
    out_ref[ER:PR, :] = jnp.zeros((PR - ER, 128), jnp.float32)


def _build_table(ef4, wblk):
    return pl.pallas_call(
        _proj_body,
        grid=(MAX_PATH_LEN,),
        in_specs=[
            pl.BlockSpec((ER, 4 * EDGE_FEAT_DIM), lambda l: (0, 0)),
            pl.BlockSpec((1, 4 * EDGE_FEAT_DIM, 128), lambda l: (l, 0, 0)),
        ],
        out_specs=pl.BlockSpec((PR, 128), lambda l: (l, 0)),
        out_shape=jax.ShapeDtypeStruct((MAX_PATH_LEN * PR, 128), jnp.float32),
    )(ef4, wblk)


@functools.partial(
    pl.kernel,
    out_type=jax.ShapeDtypeStruct((B * NUM_HEADS // 128, 128), jnp.float32),
    mesh=plsc.VectorSubcoreMesh(core_axis_name="c", subcore_axis_name="s"),
    compiler_params=pltpu.CompilerParams(use_tc_tiling_on_sc=False),
    scratch_types=[
        pltpu.VMEM((NCHUNK * MAX_PATH_LEN, CHUNK), jnp.int32),
        pltpu.VMEM((MAX_PATH_LEN * CHUNK, NUM_HEADS), jnp.float32),
        pltpu.VMEM((MAX_PATH_LEN * CHUNK, NUM_HEADS), jnp.float32),
        pltpu.VMEM((OUT_ROWS, 128), jnp.float32),
        pltpu.VMEM((OUT_ROWS, 128), jnp.float32),
        pltpu.SemaphoreType.DMA,
        pltpu.SemaphoreType.DMA,
    ],
)
def _gather_accum(table_hbm, idx_hbm, out_hbm, idx_v,
                  rows_a, rows_b, out_a, out_b, sem_a, sem_b):
    wid = lax.axis_index("s") * NUM_CORES + lax.axis_index("c")
    nrow = NCHUNK * MAX_PATH_LEN
    pltpu.sync_copy(idx_hbm.at[pl.ds(wid * nrow, nrow)], idx_v)
    out_base = wid * (PAIRS_PER_W * NUM_HEADS // 128)

    def issue(c, rows_v, sem):
        for l in range(MAX_PATH_LEN):
            pltpu.async_copy(table_hbm.at[idx_v.at[c * MAX_PATH_LEN + l]],
                             rows_v.at[pl.ds(l * CHUNK, CHUNK)], sem)

    def drain(rows_v, sem):
        pltpu.make_async_copy(
            table_hbm.at[pl.ds(0, MAX_PATH_LEN * CHUNK)], rows_v, sem).wait()

    def accum(c, rows_v, out_v):
        @plsc.parallel_loop(0, OUT_ROWS, unroll=2)
        def _(row):
            j0 = lax.shift_left(row, 2)
            for k in range(4):
                for h in range(NUM_HEADS // 16):
                    sl = pl.ds(h * 16, 16)
                    acc = rows_v[j0 + k, sl]
                    for l in range(1, MAX_PATH_LEN):
                        acc = acc + rows_v[l * CHUNK + j0 + k, sl]
                    out_v[row, pl.ds(k * 32 + h * 16, 16)] = acc

        pltpu.sync_copy(out_v, out_hbm.at[pl.ds(out_base + c * OUT_ROWS,
                                                OUT_ROWS)])

    issue(0, rows_a, sem_a)

    def body(t, carry):
        c0 = 2 * t
        issue(c0 + 1, rows_b, sem_b)
        drain(rows_a, sem_a)
        accum(c0, rows_a, out_a)

        @pl.when(t < NCHUNK // 2 - 1)
        def _():
            issue(c0 + 2, rows_a, sem_a)

        drain(rows_b, sem_b)
        accum(c0 + 1, rows_b, out_b)
        return carry

    lax.fori_loop(0, NCHUNK // 2, body, 0)


def kernel(shortest_paths, edge_feat, max_shortest_path_len, weight_embedding):
    mask = (jnp.arange(MAX_PATH_LEN)
            < jnp.minimum(MAX_PATH_LEN, max_shortest_path_len))
    w = weight_embedding[:MAX_PATH_LEN * NUM_HEADS].reshape(
        MAX_PATH_LEN, NUM_HEADS, EDGE_FEAT_DIM)
    w = w * mask.astype(w.dtype)[:, None, None]
    wblk = jax.vmap(
        lambda m: jnp.kron(jnp.eye(4, dtype=m.dtype), m.T))(w)
    ef4 = edge_feat.reshape(ER, 4 * EDGE_FEAT_DIM)

    table = _build_table(ef4, wblk).reshape(MAX_PATH_LEN * P, NUM_HEADS)

    sp = shortest_paths.reshape(B, MAX_PATH_LEN).astype(jnp.int32)
    idx = sp + (jnp.arange(MAX_PATH_LEN, dtype=jnp.int32) * P)[None, :]
    idx = (idx.reshape(NW, NCHUNK, CHUNK, MAX_PATH_LEN)
              .transpose(0, 1, 3, 2)
              .reshape(NW * NCHUNK * MAX_PATH_LEN, CHUNK))

    out = _gather_accum(table, idx)
    return out.reshape(N_NODES, N_NODES, NUM_HEADS)


# speedup vs baseline: 16.2840x; 1.0039x over previous
import functools

import jax
import jax.numpy as jnp
from jax import lax
from jax.experimental import pallas as pl
from jax.experimental.pallas import tpu as pltpu
from jax.experimental.pallas import tpu_sc as plsc

MAX_PATH_LEN = 5
EDGE_FEAT_DIM = 128
NUM_HEADS = 32
N_NODES = 256
N_EDGES = 4096
B = N_NODES * N_NODES
P = 4128
PR = P * NUM_HEADS // 128
ER = N_EDGES * NUM_HEADS // 128

NUM_CORES = 2
NUM_SUBCORES = 16
NW = NUM_CORES * NUM_SUBCORES
PAIRS_PER_W = B // NW
CHUNK = 128
NCHUNK = PAIRS_PER_W // CHUNK
OUT_ROWS = CHUNK * NUM_HEADS // 128


def _proj_body(ef4_ref, wblk_ref, out_ref):
    mm = lax.dot_general(ef4_ref[...], wblk_ref[0],
                         (((1,), (0,)), ((), ())),
                         preferred_element_type=jnp.float32)
    out_ref[0:ER, :] = mm
    out_ref[ER:PR, :] = jnp.zeros((PR - ER, 128), jnp.float32)


def _build_table(ef4, wblk):
    return pl.pallas_call(
        _proj_body,
        grid=(MAX_PATH_LEN,),
        in_specs=[
            pl.BlockSpec((ER, 4 * EDGE_FEAT_DIM), lambda l: (0, 0)),
            pl.BlockSpec((1, 4 * EDGE_FEAT_DIM, 128), lambda l: (l, 0, 0)),
        ],
        out_specs=pl.BlockSpec((PR, 128), lambda l: (l, 0)),
        out_shape=jax.ShapeDtypeStruct((MAX_PATH_LEN * PR, 128), jnp.float32),
    )(ef4, wblk)


@functools.partial(
    pl.kernel,
    out_type=jax.ShapeDtypeStruct((B * NUM_HEADS // 128, 128), jnp.float32),
    mesh=plsc.VectorSubcoreMesh(core_axis_name="c", subcore_axis_name="s"),
    compiler_params=pltpu.CompilerParams(use_tc_tiling_on_sc=False),
    scratch_types=[
        pltpu.VMEM((NCHUNK * MAX_PATH_LEN * CHUNK,), jnp.int32),
        pltpu.VMEM((MAX_PATH_LEN * CHUNK, NUM_HEADS), jnp.float32),
        pltpu.VMEM((MAX_PATH_LEN * CHUNK, NUM_HEADS), jnp.float32),
        pltpu.VMEM((OUT_ROWS, 128), jnp.float32),
        pltpu.VMEM((OUT_ROWS, 128), jnp.float32),
        pltpu.SemaphoreType.DMA,
        pltpu.SemaphoreType.DMA,
    ],
)
def _gather_accum(table_hbm, idx_hbm, out_hbm, idx_v,
                  rows_a, rows_b, out_a, out_b, sem_a, sem_b):
    wid = lax.axis_index("s") * NUM_CORES + lax.axis_index("c")
    nidx = NCHUNK * MAX_PATH_LEN * CHUNK
    pltpu.sync_copy(idx_hbm.at[pl.ds(wid * nidx, nidx)], idx_v)
    out_base = wid * (PAIRS_PER_W * NUM_HEADS // 128)
    cidx = MAX_PATH_LEN * CHUNK

    def issue(c, rows_v, sem):
        pltpu.async_copy(table_hbm.at[idx_v.at[pl.ds(c * cidx, cidx)]],
                         rows_v, sem)

    def drain(rows_v, sem):
        pltpu.make_async_copy(
            table_hbm.at[pl.ds(0, MAX_PATH_LEN * CHUNK)], rows_v, sem).wait()

    def accum(c, rows_v, out_v):
        @plsc.parallel_loop(0, OUT_ROWS, unroll=2)
        def _(row):
            j0 = lax.shift_left(row, 2)
            for k in range(4):
                for h in range(NUM_HEADS // 16):
                    sl = pl.ds(h * 16, 16)
                    acc = rows_v[j0 + k, sl]
                    for l in range(1, MAX_PATH_LEN):
                        acc = acc + rows_v[l * CHUNK + j0 + k, sl]
                    out_v[row, pl.ds(k * 32 + h * 16, 16)] = acc

        pltpu.sync_copy(out_v, out_hbm.at[pl.ds(out_base + c * OUT_ROWS,
                                                OUT_ROWS)])

    issue(0, rows_a, sem_a)

    def body(t, carry):
        c0 = 2 * t
        issue(c0 + 1, rows_b, sem_b)
        drain(rows_a, sem_a)
        accum(c0, rows_a, out_a)

        @pl.when(t < NCHUNK // 2 - 1)
        def _():
            issue(c0 + 2, rows_a, sem_a)

        drain(rows_b, sem_b)
        accum(c0 + 1, rows_b, out_b)
        return carry

    lax.fori_loop(0, NCHUNK // 2, body, 0)


def kernel(shortest_paths, edge_feat, max_shortest_path_len, weight_embedding):
    mask = (jnp.arange(MAX_PATH_LEN)
            < jnp.minimum(MAX_PATH_LEN, max_shortest_path_len))
    w = weight_embedding[:MAX_PATH_LEN * NUM_HEADS].reshape(
        MAX_PATH_LEN, NUM_HEADS, EDGE_FEAT_DIM)
    w = w * mask.astype(w.dtype)[:, None, None]
    wblk = jax.vmap(
        lambda m: jnp.kron(jnp.eye(4, dtype=m.dtype), m.T))(w)
    ef4 = edge_feat.reshape(ER, 4 * EDGE_FEAT_DIM)

    table = _build_table(ef4, wblk).reshape(MAX_PATH_LEN * P, NUM_HEADS)

    sp = shortest_paths.reshape(B, MAX_PATH_LEN).astype(jnp.int32)
    idx = sp + (jnp.arange(MAX_PATH_LEN, dtype=jnp.int32) * P)[None, :]
    idx = (idx.reshape(NW, NCHUNK, CHUNK, MAX_PATH_LEN)
              .transpose(0, 1, 3, 2)
              .reshape(NW * NCHUNK * MAX_PATH_LEN * CHUNK))

    out = _gather_accum(table, idx)
    return out.reshape(N_NODES, N_NODES, NUM_HEADS)


# speedup vs baseline: 17.0253x; 1.0455x over previous
import functools

import jax
import jax.numpy as jnp
from jax import lax
from jax.experimental import pallas as pl
from jax.experimental.pallas import tpu as pltpu
from jax.experimental.pallas import tpu_sc as plsc

MAX_PATH_LEN = 5
EDGE_FEAT_DIM = 128
NUM_HEADS = 32
N_NODES = 256
N_EDGES = 4096
B = N_NODES * N_NODES
P = 4224
PR = P * NUM_HEADS // 128
ER = N_EDGES * NUM_HEADS // 128

NUM_CORES = 2
NUM_SUBCORES = 16
NW = NUM_CORES * NUM_SUBCORES
PAIRS_PER_W = B // NW
CHUNK = 128
NCHUNK = PAIRS_PER_W // CHUNK
OUT_ROWS = CHUNK * NUM_HEADS // 128


def _proj_body(ef4_ref, wblk_ref, out_ref):
    mm = lax.dot_general(ef4_ref[...], wblk_ref[0],
                         (((1,), (0,)), ((), ())),
                         preferred_element_type=jnp.float32)
    out_ref[0:ER, :] = mm
    out_ref[ER:PR, :] = jnp.zeros((PR - ER, 128), jnp.float32)


def _build_table(ef4, wblk):
    return pl.pallas_call(
        _proj_body,
        grid=(MAX_PATH_LEN,),
        in_specs=[
            pl.BlockSpec((ER, 4 * EDGE_FEAT_DIM), lambda l: (0, 0)),
            pl.BlockSpec((1, 4 * EDGE_FEAT_DIM, 128), lambda l: (l, 0, 0)),
        ],
        out_specs=pl.BlockSpec((PR, 128), lambda l: (l, 0)),
        out_shape=jax.ShapeDtypeStruct((MAX_PATH_LEN * PR, 128), jnp.float32),
    )(ef4, wblk)


@functools.partial(
    pl.kernel,
    out_type=jax.ShapeDtypeStruct((B * NUM_HEADS // 128, 128), jnp.float32),
    mesh=plsc.VectorSubcoreMesh(core_axis_name="c", subcore_axis_name="s"),
    compiler_params=pltpu.CompilerParams(use_tc_tiling_on_sc=False),
    scratch_types=[
        pltpu.VMEM((NCHUNK * MAX_PATH_LEN * CHUNK,), jnp.int32),
        pltpu.VMEM((MAX_PATH_LEN * CHUNK, NUM_HEADS), jnp.float32),
        pltpu.VMEM((MAX_PATH_LEN * CHUNK, NUM_HEADS), jnp.float32),
        pltpu.VMEM((OUT_ROWS, 128), jnp.float32),
        pltpu.VMEM((OUT_ROWS, 128), jnp.float32),
        pltpu.VMEM_SHARED((MAX_PATH_LEN * P, NUM_HEADS), jnp.float32),
        pltpu.SemaphoreType.DMA,
        pltpu.SemaphoreType.DMA,
    ],
)
def _gather_accum(table_hbm, idx_hbm, out_hbm, idx_v,
                  rows_a, rows_b, out_a, out_b, table_sp, sem_a, sem_b):
    wid = lax.axis_index("s") * NUM_CORES + lax.axis_index("c")
    nidx = NCHUNK * MAX_PATH_LEN * CHUNK
    pltpu.sync_copy(idx_hbm.at[pl.ds(wid * nidx, nidx)], idx_v)
    sid = lax.axis_index("s")
    srows = MAX_PATH_LEN * P // NUM_SUBCORES
    pltpu.sync_copy(table_hbm.at[pl.ds(sid * srows, srows)],
                    table_sp.at[pl.ds(sid * srows, srows)])
    plsc.subcore_barrier()
    out_base = wid * (PAIRS_PER_W * NUM_HEADS // 128)
    cidx = MAX_PATH_LEN * CHUNK

    def issue(c, rows_v, sem):
        pltpu.async_copy(table_sp.at[idx_v.at[pl.ds(c * cidx, cidx)]],
                         rows_v, sem)

    def drain(rows_v, sem):
        pltpu.make_async_copy(
            table_hbm.at[pl.ds(0, MAX_PATH_LEN * CHUNK)], rows_v, sem).wait()

    def accum(c, rows_v, out_v):
        @plsc.parallel_loop(0, OUT_ROWS, unroll=2)
        def _(row):
            j0 = lax.shift_left(row, 2)
            for k in range(4):
                for h in range(NUM_HEADS // 16):
                    sl = pl.ds(h * 16, 16)
                    acc = rows_v[j0 + k, sl]
                    for l in range(1, MAX_PATH_LEN):
                        acc = acc + rows_v[l * CHUNK + j0 + k, sl]
                    out_v[row, pl.ds(k * 32 + h * 16, 16)] = acc

        pltpu.sync_copy(out_v, out_hbm.at[pl.ds(out_base + c * OUT_ROWS,
                                                OUT_ROWS)])

    issue(0, rows_a, sem_a)

    def body(t, carry):
        c0 = 2 * t
        issue(c0 + 1, rows_b, sem_b)
        drain(rows_a, sem_a)
        accum(c0, rows_a, out_a)

        @pl.when(t < NCHUNK // 2 - 1)
        def _():
            issue(c0 + 2, rows_a, sem_a)

        drain(rows_b, sem_b)
        accum(c0 + 1, rows_b, out_b)
        return carry

    lax.fori_loop(0, NCHUNK // 2, body, 0)


def kernel(shortest_paths, edge_feat, max_shortest_path_len, weight_embedding):
    mask = (jnp.arange(MAX_PATH_LEN)
            < jnp.minimum(MAX_PATH_LEN, max_shortest_path_len))
    w = weight_embedding[:MAX_PATH_LEN * NUM_HEADS].reshape(
        MAX_PATH_LEN, NUM_HEADS, EDGE_FEAT_DIM)
    w = w * mask.astype(w.dtype)[:, None, None]
    wblk = jax.vmap(
        lambda m: jnp.kron(jnp.eye(4, dtype=m.dtype), m.T))(w)
    ef4 = edge_feat.reshape(ER, 4 * EDGE_FEAT_DIM)

    table = _build_table(ef4, wblk).reshape(MAX_PATH_LEN * P, NUM_HEADS)

    sp = shortest_paths.reshape(B, MAX_PATH_LEN).astype(jnp.int32)
    idx = sp + (jnp.arange(MAX_PATH_LEN, dtype=jnp.int32) * P)[None, :]
    idx = (idx.reshape(NW, NCHUNK, CHUNK, MAX_PATH_LEN)
              .transpose(0, 1, 3, 2)
              .reshape(NW * NCHUNK * MAX_PATH_LEN * CHUNK))

    out = _gather_accum(table, idx)
    return out.reshape(N_NODES, N_NODES, NUM_HEADS)
